# baseline (speedup 1.0000x reference)
import functools

import jax
import jax.numpy as jnp
from jax import lax
from jax.experimental import pallas as pl
from jax.experimental.pallas import tpu as pltpu
from jax.experimental.pallas import tpu_sc as plsc

N_USER = 25000
N_ITEM = 25000
N = N_USER + N_ITEM
E = 800000
D = 96
NPART = 6
PARTS_PER_CORE = 3
NC = 2
NS = 16
LANES = 16

EDGES_PER_TILE = E // NS
SUBW = 80
SUB = 5
BLK_E = SUB * SUBW
NBLOCK = EDGES_PER_TILE // BLK_E
ROWS80_PER_TILE = EDGES_PER_TILE // SUBW
ROWS_PER_TILE = N // NS

_mesh = plsc.VectorSubcoreMesh(core_axis_name="c", subcore_axis_name="s")
_sc_params = pltpu.CompilerParams(
    use_tc_tiling_on_sc=False, needs_layout_passes=False)


@functools.partial(
    pl.kernel,
    out_type=jax.ShapeDtypeStruct((N, D), jnp.float32),
    mesh=_mesh,
    compiler_params=_sc_params,
    scratch_types=[
        pltpu.VMEM_SHARED((N, LANES), jnp.float32),
        pltpu.VMEM_SHARED((N, LANES), jnp.float32),
        [pltpu.VMEM((BLK_E, LANES), jnp.float32) for _ in range(3)],
        [pltpu.VMEM((SUB, SUBW), jnp.int32) for _ in range(3)],
        [pltpu.VMEM((SUB, SUBW), jnp.int32) for _ in range(3)],
        [pltpu.VMEM((BLK_E,), jnp.float32) for _ in range(3)],
        [pltpu.SemaphoreType.DMA for _ in range(3)],
        [pltpu.SemaphoreType.DMA for _ in range(3)],
    ],
)
def _spmm_sc(x2d, rows2d, cols2d, vals_hbm, out_hbm,
             acc, xloc, G, CI, RI, VB, semE, semG):
    c = lax.axis_index("c")
    s = lax.axis_index("s")

    def eload_at(o, b):
        vbase = s * EDGES_PER_TILE + b * BLK_E
        rbase = s * ROWS80_PER_TILE + b * SUB
        return [
            (vals_hbm.at[pl.ds(vbase, BLK_E)], VB[o]),
            (rows2d.at[pl.ds(rbase, SUB)], RI[o]),
            (cols2d.at[pl.ds(rbase, SUB)], CI[o]),
        ]

    def eload(o, b):
        for src, dst in eload_at(o, b):
            pltpu.async_copy(src, dst, semE[o])

    def ewait(o, b):
        for src, dst in eload_at(o, b):
            pltpu.make_async_copy(src, dst, semE[o]).wait()

    def gfire(o):
        for j in range(SUB):
            pltpu.async_copy(xloc.at[CI[o].at[j]],
                             G[o].at[pl.ds(j * SUBW, SUBW)], semG[o])

    def gwait(o):
        for j in range(SUB):
            pltpu.make_async_copy(xloc.at[CI[o].at[j]],
                                  G[o].at[pl.ds(j * SUBW, SUBW)],
                                  semG[o]).wait()

    def scale(o):
        g = G[o]
        vb = VB[o]

        @plsc.parallel_loop(0, BLK_E, step=1, unroll=8)
        def _(e):
            bval = plsc.load_gather(vb, [jnp.full((LANES,), e, jnp.int32)])
            g[e, :] = g[e, :] * bval

    def scat(o):
        for j in range(SUB):
            pltpu.sync_copy(G[o].at[pl.ds(j * SUBW, SUBW)],
                            acc.at[RI[o].at[j]], add=True)

    for p_local in range(PARTS_PER_CORE):
        p = c * PARTS_PER_CORE + p_local

        def zfill(i, _):
            G[0][i, :] = jnp.zeros((LANES,), jnp.float32)
            return 0
        lax.fori_loop(0, BLK_E, zfill, 0)
        for z in range(7):
            pltpu.sync_copy(
                G[0], acc.at[pl.ds(s * ROWS_PER_TILE + z * BLK_E, BLK_E)])
        pltpu.sync_copy(
            G[0].at[pl.ds(0, ROWS_PER_TILE - 7 * BLK_E)],
            acc.at[pl.ds(s * ROWS_PER_TILE + 7 * BLK_E,
                         ROWS_PER_TILE - 7 * BLK_E)])
        pltpu.sync_copy(
            x2d.at[pl.ds(s * ROWS_PER_TILE, ROWS_PER_TILE),
                   pl.ds(p * LANES, LANES)],
            xloc.at[pl.ds(s * ROWS_PER_TILE, ROWS_PER_TILE)])
        plsc.subcore_barrier()

        eload(0, 0)
        eload(1, 1)
        ewait(0, 0)
        gfire(0)

        def steady(m, _):
            for o in range(3):
                b = 3 * m + o
                o2 = (o + 2) % 3
                o1 = (o + 1) % 3

                eload(o2, b + 2)
                ewait(o1, b + 1)
                gfire(o1)
                gwait(o)
                scale(o)
                scat(o)
            return 0
        lax.fori_loop(0, (NBLOCK - 2) // 3, steady, 0)

        ewait(1, NBLOCK - 1)
        gfire(1)
        gwait(0)
        scale(0)
        scat(0)
        gwait(1)
        scale(1)
        scat(1)
        plsc.subcore_barrier()

        pltpu.sync_copy(acc.at[pl.ds(s * ROWS_PER_TILE, ROWS_PER_TILE)],
                        out_hbm.at[pl.ds(s * ROWS_PER_TILE, ROWS_PER_TILE),
                                   pl.ds(p * LANES, LANES)])
        plsc.subcore_barrier()


def _spmm(x, rows, cols, vals):
    return _spmm_sc(x, rows.reshape(E // SUBW, SUBW),
                    cols.reshape(E // SUBW, SUBW), vals)


BLK = 400
NBLK = N // BLK


def _layer_tc_body(alpha_ref, ego_ref, h1_ref, h2_ref, h3_ref,
                   wgc_ref, bgc_ref, wbi_ref, bbi_ref, act_ref, out_ref):
    a0 = alpha_ref[0, 0]
    a1 = alpha_ref[0, 1]
    a2 = alpha_ref[0, 2]
    m = jnp.maximum(jnp.maximum(a0, a1), a2)
    e0 = jnp.exp(a0 - m)
    e1 = jnp.exp(a1 - m)
    e2 = jnp.exp(a2 - m)
    tot = e0 + e1 + e2
    b0 = e0 / tot
    b1 = e1 / tot
    b2 = e2 / tot
    side = b0 * h1_ref[...] + b1 * h2_ref[...] + b2 * h3_ref[...]
    ego = ego_ref[...]
    sum_e = jnp.dot(side, wgc_ref[...], preferred_element_type=jnp.float32) + bgc_ref[...]
    bi = jnp.dot(ego * side, wbi_ref[...], preferred_element_type=jnp.float32) + bbi_ref[...]
    act = jnp.where(sum_e >= 0, sum_e, 0.2 * sum_e) + bi
    act_ref[...] = act
    nrm = jnp.sqrt(jnp.sum(act * act, axis=1, keepdims=True))
    out_ref[...] = act / jnp.maximum(nrm, 1e-12)


def _layer_tc(alpha, ego, h1, h2, h3, wgc, bgc, wbi, bbi):
    return pl.pallas_call(
        _layer_tc_body,
        grid=(NBLK,),
        in_specs=[
            pl.BlockSpec(memory_space=pltpu.SMEM),
            pl.BlockSpec((BLK, D), lambda i: (i, 0)),
            pl.BlockSpec((BLK, D), lambda i: (i, 0)),
            pl.BlockSpec((BLK, D), lambda i: (i, 0)),
            pl.BlockSpec((BLK, D), lambda i: (i, 0)),
            pl.BlockSpec((D, D), lambda i: (0, 0)),
            pl.BlockSpec((1, D), lambda i: (0, 0)),
            pl.BlockSpec((D, D), lambda i: (0, 0)),
            pl.BlockSpec((1, D), lambda i: (0, 0)),
        ],
        out_specs=[pl.BlockSpec((BLK, D), lambda i: (i, 0)),
                   pl.BlockSpec((BLK, D), lambda i: (i, 0))],
        out_shape=[jax.ShapeDtypeStruct((N, D), jnp.float32),
                   jax.ShapeDtypeStruct((N, D), jnp.float32)],
    )(alpha.reshape(1, 3), ego, h1, h2, h3, wgc, bgc, wbi, bbi)


B3 = 3072
B_PER_W = B3 // (NC * NS)


@functools.partial(
    pl.kernel,
    out_type=[jax.ShapeDtypeStruct((B3, D), jnp.float32)] * 3,
    mesh=_mesh,
    compiler_params=_sc_params,
    scratch_types=[
        pltpu.VMEM((B_PER_W,), jnp.int32),
        pltpu.VMEM((B_PER_W, D), jnp.float32),
        pltpu.VMEM((B_PER_W, D), jnp.float32),
        pltpu.VMEM((B_PER_W, D), jnp.float32),
        pltpu.SemaphoreType.DMA,
    ],
)
def _batch_gather_sc(idx_hbm, t0, t1, t2, o0, o1, o2,
                     idx_v, r0, r1, r2, sem):
    c = lax.axis_index("c")
    s = lax.axis_index("s")
    wid = s * NC + c
    base = wid * B_PER_W
    pltpu.sync_copy(idx_hbm.at[pl.ds(base, B_PER_W)], idx_v)
    cp0 = pltpu.async_copy(t0.at[idx_v], r0, sem)
    cp1 = pltpu.async_copy(t1.at[idx_v], r1, sem)
    cp2 = pltpu.async_copy(t2.at[idx_v], r2, sem)
    cp0.wait()
    cp1.wait()
    cp2.wait()
    pltpu.sync_copy(r0, o0.at[pl.ds(base, B_PER_W)])
    pltpu.sync_copy(r1, o1.at[pl.ds(base, B_PER_W)])
    pltpu.sync_copy(r2, o2.at[pl.ds(base, B_PER_W)])


@jax.jit
def kernel(adj_rows, adj_cols, adj_vals, feature_dense, user_emb, item_emb1,
           item_emb2, alpha0, alpha1, W_gc_0, b_gc_0, W_bi_0, b_bi_0,
           W_gc_1, b_gc_1, W_bi_1, b_bi_1, users, pos_items, neg_items):
    ego0 = jnp.concatenate(
        [jnp.concatenate([user_emb, item_emb1], axis=0),
         jnp.concatenate([feature_dense, item_emb2], axis=0)], axis=1)

    alphas = [alpha0, alpha1]
    weights = [(W_gc_0, b_gc_0, W_bi_0, b_bi_0),
               (W_gc_1, b_gc_1, W_bi_1, b_bi_1)]
    ego = ego0
    embs = [ego0]
    for k in range(2):
        h1 = _spmm(ego, adj_rows, adj_cols, adj_vals)
        h2 = _spmm(h1, adj_rows, adj_cols, adj_vals)
        h3 = _spmm(h2, adj_rows, adj_cols, adj_vals)
        wgc, bgc, wbi, bbi = weights[k]
        ego, norm = _layer_tc(alphas[k], ego, h1, h2, h3, wgc, bgc, wbi, bbi)
        embs.append(norm)

    idx3 = jnp.concatenate(
        [users, N_USER + pos_items, N_USER + neg_items], axis=0)
    o0, o1, o2 = _batch_gather_sc(idx3, embs[0], embs[1], embs[2])
    return jnp.concatenate([o0, o1, o2], axis=1)


# speedup vs baseline: 1.1056x; 1.1056x over previous
import functools

import jax
import jax.numpy as jnp
from jax import lax
from jax.experimental import pallas as pl
from jax.experimental.pallas import tpu as pltpu
from jax.experimental.pallas import tpu_sc as plsc

N_USER = 25000
N_ITEM = 25000
N = N_USER + N_ITEM
E = 800000
D = 96
NPART = 6
PARTS_PER_CORE = 3
NC = 2
NS = 16
LANES = 16

EDGES_PER_TILE = E // NS
SUBW = 80
SUB = 5
BLK_E = SUB * SUBW
NBLOCK = EDGES_PER_TILE // BLK_E
ROWS80_PER_TILE = EDGES_PER_TILE // SUBW
ROWS_PER_TILE = N // NS

_mesh = plsc.VectorSubcoreMesh(core_axis_name="c", subcore_axis_name="s")
_sc_params = pltpu.CompilerParams(
    use_tc_tiling_on_sc=False, needs_layout_passes=False)


@functools.partial(
    pl.kernel,
    out_type=jax.ShapeDtypeStruct((N, D), jnp.float32),
    mesh=_mesh,
    compiler_params=_sc_params,
    scratch_types=[
        pltpu.VMEM_SHARED((N, LANES), jnp.float32),
        pltpu.VMEM_SHARED((N, LANES), jnp.float32),
        [pltpu.VMEM((BLK_E, LANES), jnp.float32) for _ in range(3)],
        [pltpu.VMEM((BLK_E,), jnp.int32) for _ in range(3)],
        [pltpu.VMEM((BLK_E,), jnp.int32) for _ in range(3)],
        [pltpu.VMEM((BLK_E,), jnp.float32) for _ in range(3)],
        [pltpu.SemaphoreType.DMA for _ in range(3)],
        [pltpu.SemaphoreType.DMA for _ in range(3)],
    ],
)
def _spmm_sc(x2d, rows_hbm, cols_hbm, vals_hbm, out_hbm,
             acc, xloc, G, CI, RI, VB, semE, semG):
    c = lax.axis_index("c")
    s = lax.axis_index("s")

    def eload_at(o, b):
        base = s * EDGES_PER_TILE + b * BLK_E
        return [
            (vals_hbm.at[pl.ds(base, BLK_E)], VB[o]),
            (rows_hbm.at[pl.ds(base, BLK_E)], RI[o]),
            (cols_hbm.at[pl.ds(base, BLK_E)], CI[o]),
        ]

    def eload(o, b):
        for src, dst in eload_at(o, b):
            pltpu.async_copy(src, dst, semE[o])

    def ewait(o, b):
        for src, dst in eload_at(o, b):
            pltpu.make_async_copy(src, dst, semE[o]).wait()

    def gfire(o):
        pltpu.async_copy(xloc.at[CI[o]], G[o], semG[o])

    def gwait(o):
        pltpu.make_async_copy(xloc.at[CI[o]], G[o], semG[o]).wait()

    def scale(o):
        g = G[o]
        vb = VB[o]

        @plsc.parallel_loop(0, BLK_E, step=1, unroll=8)
        def _(e):
            bval = plsc.load_gather(vb, [jnp.full((LANES,), e, jnp.int32)])
            g[e, :] = g[e, :] * bval

    def scat(o):
        pltpu.sync_copy(G[o], acc.at[RI[o]], add=True)

    for p_local in range(PARTS_PER_CORE):
        p = c * PARTS_PER_CORE + p_local

        def zfill(i, _):
            G[0][i, :] = jnp.zeros((LANES,), jnp.float32)
            return 0
        lax.fori_loop(0, BLK_E, zfill, 0)
        for z in range(7):
            pltpu.sync_copy(
                G[0], acc.at[pl.ds(s * ROWS_PER_TILE + z * BLK_E, BLK_E)])
        pltpu.sync_copy(
            G[0].at[pl.ds(0, ROWS_PER_TILE - 7 * BLK_E)],
            acc.at[pl.ds(s * ROWS_PER_TILE + 7 * BLK_E,
                         ROWS_PER_TILE - 7 * BLK_E)])
        pltpu.sync_copy(
            x2d.at[pl.ds(s * ROWS_PER_TILE, ROWS_PER_TILE),
                   pl.ds(p * LANES, LANES)],
            xloc.at[pl.ds(s * ROWS_PER_TILE, ROWS_PER_TILE)])
        plsc.subcore_barrier()

        eload(0, 0)
        eload(1, 1)
        ewait(0, 0)
        gfire(0)

        def steady(m, _):
            for o in range(3):
                b = 3 * m + o
                o2 = (o + 2) % 3
                o1 = (o + 1) % 3

                eload(o2, b + 2)
                ewait(o1, b + 1)
                gfire(o1)
                gwait(o)
                scale(o)
                scat(o)
            return 0
        lax.fori_loop(0, (NBLOCK - 2) // 3, steady, 0)

        ewait(1, NBLOCK - 1)
        gfire(1)
        gwait(0)
        scale(0)
        scat(0)
        gwait(1)
        scale(1)
        scat(1)
        plsc.subcore_barrier()

        pltpu.sync_copy(acc.at[pl.ds(s * ROWS_PER_TILE, ROWS_PER_TILE)],
                        out_hbm.at[pl.ds(s * ROWS_PER_TILE, ROWS_PER_TILE),
                                   pl.ds(p * LANES, LANES)])
        plsc.subcore_barrier()


def _spmm(x, rows, cols, vals):
    return _spmm_sc(x, rows, cols, vals)


BLK = 400
NBLK = N // BLK


def _layer_tc_body(alpha_ref, ego_ref, h1_ref, h2_ref, h3_ref,
                   wgc_ref, bgc_ref, wbi_ref, bbi_ref, act_ref, out_ref):
    a0 = alpha_ref[0, 0]
    a1 = alpha_ref[0, 1]
    a2 = alpha_ref[0, 2]
    m = jnp.maximum(jnp.maximum(a0, a1), a2)
    e0 = jnp.exp(a0 - m)
    e1 = jnp.exp(a1 - m)
    e2 = jnp.exp(a2 - m)
    tot = e0 + e1 + e2
    b0 = e0 / tot
    b1 = e1 / tot
    b2 = e2 / tot
    side = b0 * h1_ref[...] + b1 * h2_ref[...] + b2 * h3_ref[...]
    ego = ego_ref[...]
    sum_e = jnp.dot(side, wgc_ref[...], preferred_element_type=jnp.float32) + bgc_ref[...]
    bi = jnp.dot(ego * side, wbi_ref[...], preferred_element_type=jnp.float32) + bbi_ref[...]
    act = jnp.where(sum_e >= 0, sum_e, 0.2 * sum_e) + bi
    act_ref[...] = act
    nrm = jnp.sqrt(jnp.sum(act * act, axis=1, keepdims=True))
    out_ref[...] = act / jnp.maximum(nrm, 1e-12)


def _layer_tc(alpha, ego, h1, h2, h3, wgc, bgc, wbi, bbi):
    return pl.pallas_call(
        _layer_tc_body,
        grid=(NBLK,),
        in_specs=[
            pl.BlockSpec(memory_space=pltpu.SMEM),
            pl.BlockSpec((BLK, D), lambda i: (i, 0)),
            pl.BlockSpec((BLK, D), lambda i: (i, 0)),
            pl.BlockSpec((BLK, D), lambda i: (i, 0)),
            pl.BlockSpec((BLK, D), lambda i: (i, 0)),
            pl.BlockSpec((D, D), lambda i: (0, 0)),
            pl.BlockSpec((1, D), lambda i: (0, 0)),
            pl.BlockSpec((D, D), lambda i: (0, 0)),
            pl.BlockSpec((1, D), lambda i: (0, 0)),
        ],
        out_specs=[pl.BlockSpec((BLK, D), lambda i: (i, 0)),
                   pl.BlockSpec((BLK, D), lambda i: (i, 0))],
        out_shape=[jax.ShapeDtypeStruct((N, D), jnp.float32),
                   jax.ShapeDtypeStruct((N, D), jnp.float32)],
    )(alpha.reshape(1, 3), ego, h1, h2, h3, wgc, bgc, wbi, bbi)


B3 = 3072
B_PER_W = B3 // (NC * NS)


@functools.partial(
    pl.kernel,
    out_type=[jax.ShapeDtypeStruct((B3, D), jnp.float32)] * 3,
    mesh=_mesh,
    compiler_params=_sc_params,
    scratch_types=[
        pltpu.VMEM((B_PER_W,), jnp.int32),
        pltpu.VMEM((B_PER_W, D), jnp.float32),
        pltpu.VMEM((B_PER_W, D), jnp.float32),
        pltpu.VMEM((B_PER_W, D), jnp.float32),
        pltpu.SemaphoreType.DMA,
    ],
)
def _batch_gather_sc(idx_hbm, t0, t1, t2, o0, o1, o2,
                     idx_v, r0, r1, r2, sem):
    c = lax.axis_index("c")
    s = lax.axis_index("s")
    wid = s * NC + c
    base = wid * B_PER_W
    pltpu.sync_copy(idx_hbm.at[pl.ds(base, B_PER_W)], idx_v)
    cp0 = pltpu.async_copy(t0.at[idx_v], r0, sem)
    cp1 = pltpu.async_copy(t1.at[idx_v], r1, sem)
    cp2 = pltpu.async_copy(t2.at[idx_v], r2, sem)
    cp0.wait()
    cp1.wait()
    cp2.wait()
    pltpu.sync_copy(r0, o0.at[pl.ds(base, B_PER_W)])
    pltpu.sync_copy(r1, o1.at[pl.ds(base, B_PER_W)])
    pltpu.sync_copy(r2, o2.at[pl.ds(base, B_PER_W)])


@jax.jit
def kernel(adj_rows, adj_cols, adj_vals, feature_dense, user_emb, item_emb1,
           item_emb2, alpha0, alpha1, W_gc_0, b_gc_0, W_bi_0, b_bi_0,
           W_gc_1, b_gc_1, W_bi_1, b_bi_1, users, pos_items, neg_items):
    ego0 = jnp.concatenate(
        [jnp.concatenate([user_emb, item_emb1], axis=0),
         jnp.concatenate([feature_dense, item_emb2], axis=0)], axis=1)

    alphas = [alpha0, alpha1]
    weights = [(W_gc_0, b_gc_0, W_bi_0, b_bi_0),
               (W_gc_1, b_gc_1, W_bi_1, b_bi_1)]
    ego = ego0
    embs = [ego0]
    for k in range(2):
        h1 = _spmm(ego, adj_rows, adj_cols, adj_vals)
        h2 = _spmm(h1, adj_rows, adj_cols, adj_vals)
        h3 = _spmm(h2, adj_rows, adj_cols, adj_vals)
        wgc, bgc, wbi, bbi = weights[k]
        ego, norm = _layer_tc(alphas[k], ego, h1, h2, h3, wgc, bgc, wbi, bbi)
        embs.append(norm)

    idx3 = jnp.concatenate(
        [users, N_USER + pos_items, N_USER + neg_items], axis=0)
    o0, o1, o2 = _batch_gather_sc(idx3, embs[0], embs[1], embs[2])
    return jnp.concatenate([o0, o1, o2], axis=1)


# speedup vs baseline: 1.3581x; 1.2284x over previous
import functools

import jax
import jax.numpy as jnp
from jax import lax
from jax.experimental import pallas as pl
from jax.experimental.pallas import tpu as pltpu
from jax.experimental.pallas import tpu_sc as plsc

N_USER = 25000
N_ITEM = 25000
N = N_USER + N_ITEM
E = 800000
D = 96
NPART = 6
PARTS_PER_CORE = 3
NC = 2
NS = 16
LANES = 16

EDGES_PER_TILE = E // NS
SUBW = 80
SUB = 5
BLK_E = SUB * SUBW
NBLOCK = EDGES_PER_TILE // BLK_E
ROWS80_PER_TILE = EDGES_PER_TILE // SUBW
ROWS_PER_TILE = N // NS

_mesh = plsc.VectorSubcoreMesh(core_axis_name="c", subcore_axis_name="s")
_sc_params = pltpu.CompilerParams(
    use_tc_tiling_on_sc=False, needs_layout_passes=False)


@functools.partial(
    pl.kernel,
    out_type=jax.ShapeDtypeStruct((N, D), jnp.float32),
    mesh=_mesh,
    compiler_params=_sc_params,
    scratch_types=[
        pltpu.VMEM_SHARED((N, LANES), jnp.float32),
        [pltpu.VMEM((BLK_E, LANES), jnp.float32) for _ in range(4)],
        [pltpu.VMEM((BLK_E,), jnp.int32) for _ in range(4)],
        [pltpu.VMEM((BLK_E,), jnp.int32) for _ in range(4)],
        [pltpu.VMEM((BLK_E,), jnp.float32) for _ in range(4)],
        [pltpu.SemaphoreType.DMA for _ in range(4)],
        [pltpu.SemaphoreType.DMA for _ in range(4)],
        [pltpu.SemaphoreType.DMA for _ in range(4)],
    ],
)
def _spmm_sc(x_flat, rows_hbm, cols_hbm, vals_hbm, out_hbm,
             acc, G, CI, RI, VB, semE, semG, semS):
    c = lax.axis_index("c")
    s = lax.axis_index("s")

    def eload_at(o, b):
        base = s * EDGES_PER_TILE + b * BLK_E
        return [
            (vals_hbm.at[pl.ds(base, BLK_E)], VB[o]),
            (rows_hbm.at[pl.ds(base, BLK_E)], RI[o]),
            (cols_hbm.at[pl.ds(base, BLK_E)], CI[o]),
        ]

    def eload(o, b):
        for src, dst in eload_at(o, b):
            pltpu.async_copy(src, dst, semE[o])

    def ewait(o, b):
        for src, dst in eload_at(o, b):
            pltpu.make_async_copy(src, dst, semE[o]).wait()

    def transform(o, p):
        ci = CI[o]

        def tfb(i, _):
            sl = pl.ds(i * LANES, LANES)
            ci[sl] = ci[sl] * NPART + p
            return 0
        lax.fori_loop(0, BLK_E // LANES, tfb, 0)

    def gfire(o):
        pltpu.async_copy(x_flat.at[CI[o]], G[o], semG[o])

    def gwait(o):
        pltpu.make_async_copy(x_flat.at[CI[o]], G[o], semG[o]).wait()

    def scale(o):
        g = G[o]
        vb = VB[o]

        @plsc.parallel_loop(0, BLK_E, step=1, unroll=8)
        def _(e):
            bval = plsc.load_gather(vb, [jnp.full((LANES,), e, jnp.int32)])
            g[e, :] = g[e, :] * bval

    def scat_fire(o):
        pltpu.async_copy(G[o], acc.at[RI[o]], semS[o], add=True)

    def scat_wait(o):
        pltpu.make_async_copy(G[o], acc.at[RI[o]], semS[o]).wait()

    for p_local in range(PARTS_PER_CORE):
        p = c * PARTS_PER_CORE + p_local

        def zfill(i, _):
            G[0][i, :] = jnp.zeros((LANES,), jnp.float32)
            return 0
        lax.fori_loop(0, BLK_E, zfill, 0)
        for z in range(7):
            pltpu.sync_copy(
                G[0], acc.at[pl.ds(s * ROWS_PER_TILE + z * BLK_E, BLK_E)])
        pltpu.sync_copy(
            G[0].at[pl.ds(0, ROWS_PER_TILE - 7 * BLK_E)],
            acc.at[pl.ds(s * ROWS_PER_TILE + 7 * BLK_E,
                         ROWS_PER_TILE - 7 * BLK_E)])
        plsc.subcore_barrier()

        def position(b, o, warm):
            o1 = (b + 1) % 4
            o2 = (b + 2) % 4
            if warm:
                scat_wait(o2)
            if b + 2 < NBLOCK:
                eload(o2, b + 2)
            if b + 1 < NBLOCK:
                ewait(o1, b + 1)
                transform(o1, p)
                gfire(o1)
            gwait(o)
            scale(o)
            scat_fire(o)

        eload(0, 0)
        eload(1, 1)
        ewait(0, 0)
        transform(0, p)
        gfire(0)
        for b in range(4):
            position(b, b % 4, b >= 2)

        def steady(m, _):
            for o in range(4):
                b = 4 + 4 * m + o
                scat_wait((o + 2) % 4)
                eload((o + 2) % 4, b + 2)
                ewait((o + 1) % 4, b + 1)
                transform((o + 1) % 4, p)
                gfire((o + 1) % 4)
                gwait(o)
                scale(o)
                scat_fire(o)
            return 0
        lax.fori_loop(0, (NBLOCK - 4 - 5) // 4, steady, 0)

        for b in range(NBLOCK - 5, NBLOCK):
            position(b, b % 4, True)
        scat_wait((NBLOCK - 2) % 4)
        scat_wait((NBLOCK - 1) % 4)
        plsc.subcore_barrier()

        pltpu.sync_copy(acc.at[pl.ds(s * ROWS_PER_TILE, ROWS_PER_TILE)],
                        out_hbm.at[pl.ds(s * ROWS_PER_TILE, ROWS_PER_TILE),
                                   pl.ds(p * LANES, LANES)])
        plsc.subcore_barrier()


def _spmm(x, rows, cols, vals):
    return _spmm_sc(x.reshape(N * NPART, LANES), rows, cols, vals)


BLK = 400
NBLK = N // BLK


def _layer_tc_body(alpha_ref, ego_ref, h1_ref, h2_ref, h3_ref,
                   wgc_ref, bgc_ref, wbi_ref, bbi_ref, act_ref, out_ref):
    a0 = alpha_ref[0, 0]
    a1 = alpha_ref[0, 1]
    a2 = alpha_ref[0, 2]
    m = jnp.maximum(jnp.maximum(a0, a1), a2)
    e0 = jnp.exp(a0 - m)
    e1 = jnp.exp(a1 - m)
    e2 = jnp.exp(a2 - m)
    tot = e0 + e1 + e2
    b0 = e0 / tot
    b1 = e1 / tot
    b2 = e2 / tot
    side = b0 * h1_ref[...] + b1 * h2_ref[...] + b2 * h3_ref[...]
    ego = ego_ref[...]
    sum_e = jnp.dot(side, wgc_ref[...], preferred_element_type=jnp.float32) + bgc_ref[...]
    bi = jnp.dot(ego * side, wbi_ref[...], preferred_element_type=jnp.float32) + bbi_ref[...]
    act = jnp.where(sum_e >= 0, sum_e, 0.2 * sum_e) + bi
    act_ref[...] = act
    nrm = jnp.sqrt(jnp.sum(act * act, axis=1, keepdims=True))
    out_ref[...] = act / jnp.maximum(nrm, 1e-12)


def _layer_tc(alpha, ego, h1, h2, h3, wgc, bgc, wbi, bbi):
    return pl.pallas_call(
        _layer_tc_body,
        grid=(NBLK,),
        in_specs=[
            pl.BlockSpec(memory_space=pltpu.SMEM),
            pl.BlockSpec((BLK, D), lambda i: (i, 0)),
            pl.BlockSpec((BLK, D), lambda i: (i, 0)),
            pl.BlockSpec((BLK, D), lambda i: (i, 0)),
            pl.BlockSpec((BLK, D), lambda i: (i, 0)),
            pl.BlockSpec((D, D), lambda i: (0, 0)),
            pl.BlockSpec((1, D), lambda i: (0, 0)),
            pl.BlockSpec((D, D), lambda i: (0, 0)),
            pl.BlockSpec((1, D), lambda i: (0, 0)),
        ],
        out_specs=[pl.BlockSpec((BLK, D), lambda i: (i, 0)),
                   pl.BlockSpec((BLK, D), lambda i: (i, 0))],
        out_shape=[jax.ShapeDtypeStruct((N, D), jnp.float32),
                   jax.ShapeDtypeStruct((N, D), jnp.float32)],
    )(alpha.reshape(1, 3), ego, h1, h2, h3, wgc, bgc, wbi, bbi)


B3 = 3072
B_PER_W = B3 // (NC * NS)


@functools.partial(
    pl.kernel,
    out_type=[jax.ShapeDtypeStruct((B3, D), jnp.float32)] * 3,
    mesh=_mesh,
    compiler_params=_sc_params,
    scratch_types=[
        pltpu.VMEM((B_PER_W,), jnp.int32),
        pltpu.VMEM((B_PER_W, D), jnp.float32),
        pltpu.VMEM((B_PER_W, D), jnp.float32),
        pltpu.VMEM((B_PER_W, D), jnp.float32),
        pltpu.SemaphoreType.DMA,
    ],
)
def _batch_gather_sc(idx_hbm, t0, t1, t2, o0, o1, o2,
                     idx_v, r0, r1, r2, sem):
    c = lax.axis_index("c")
    s = lax.axis_index("s")
    wid = s * NC + c
    base = wid * B_PER_W
    pltpu.sync_copy(idx_hbm.at[pl.ds(base, B_PER_W)], idx_v)
    cp0 = pltpu.async_copy(t0.at[idx_v], r0, sem)
    cp1 = pltpu.async_copy(t1.at[idx_v], r1, sem)
    cp2 = pltpu.async_copy(t2.at[idx_v], r2, sem)
    cp0.wait()
    cp1.wait()
    cp2.wait()
    pltpu.sync_copy(r0, o0.at[pl.ds(base, B_PER_W)])
    pltpu.sync_copy(r1, o1.at[pl.ds(base, B_PER_W)])
    pltpu.sync_copy(r2, o2.at[pl.ds(base, B_PER_W)])


@jax.jit
def kernel(adj_rows, adj_cols, adj_vals, feature_dense, user_emb, item_emb1,
           item_emb2, alpha0, alpha1, W_gc_0, b_gc_0, W_bi_0, b_bi_0,
           W_gc_1, b_gc_1, W_bi_1, b_bi_1, users, pos_items, neg_items):
    ego0 = jnp.concatenate(
        [jnp.concatenate([user_emb, item_emb1], axis=0),
         jnp.concatenate([feature_dense, item_emb2], axis=0)], axis=1)

    alphas = [alpha0, alpha1]
    weights = [(W_gc_0, b_gc_0, W_bi_0, b_bi_0),
               (W_gc_1, b_gc_1, W_bi_1, b_bi_1)]
    ego = ego0
    embs = [ego0]
    for k in range(2):
        h1 = _spmm(ego, adj_rows, adj_cols, adj_vals)
        h2 = _spmm(h1, adj_rows, adj_cols, adj_vals)
        h3 = _spmm(h2, adj_rows, adj_cols, adj_vals)
        wgc, bgc, wbi, bbi = weights[k]
        ego, norm = _layer_tc(alphas[k], ego, h1, h2, h3, wgc, bgc, wbi, bbi)
        embs.append(norm)

    idx3 = jnp.concatenate(
        [users, N_USER + pos_items, N_USER + neg_items], axis=0)
    o0, o1, o2 = _batch_gather_sc(idx3, embs[0], embs[1], embs[2])
    return jnp.concatenate([o0, o1, o2], axis=1)


# speedup vs baseline: 1.5542x; 1.1444x over previous
import functools

import jax
import jax.numpy as jnp
from jax import lax
from jax.experimental import pallas as pl
from jax.experimental.pallas import tpu as pltpu
from jax.experimental.pallas import tpu_sc as plsc

N_USER = 25000
N_ITEM = 25000
N = N_USER + N_ITEM
E = 800000
D = 96
NPART = 6
PARTS_PER_CORE = 3
NC = 2
NS = 16
LANES = 16

EDGES_PER_TILE = E // NS
BLK_E = 1000
NBLOCK = EDGES_PER_TILE // BLK_E
TAIL = ((NBLOCK - 4 - 2) % 4) + 2
STEADY = (NBLOCK - 4 - TAIL) // 4
ROWS_PER_TILE = N // NS
ZCOPIES = ROWS_PER_TILE // BLK_E
ZREM = ROWS_PER_TILE - ZCOPIES * BLK_E

_mesh = plsc.VectorSubcoreMesh(core_axis_name="c", subcore_axis_name="s")
_sc_params = pltpu.CompilerParams(
    use_tc_tiling_on_sc=False, needs_layout_passes=False)


@functools.partial(
    pl.kernel,
    out_type=jax.ShapeDtypeStruct((N, D), jnp.float32),
    mesh=_mesh,
    compiler_params=_sc_params,
    scratch_types=[
        pltpu.VMEM_SHARED((N, LANES), jnp.float32),
        [pltpu.VMEM((BLK_E, LANES), jnp.float32) for _ in range(4)],
        [pltpu.VMEM((BLK_E,), jnp.int32) for _ in range(4)],
        [pltpu.VMEM((BLK_E,), jnp.int32) for _ in range(4)],
        [pltpu.VMEM((BLK_E,), jnp.float32) for _ in range(4)],
        [pltpu.SemaphoreType.DMA for _ in range(4)],
        [pltpu.SemaphoreType.DMA for _ in range(4)],
        [pltpu.SemaphoreType.DMA for _ in range(4)],
    ],
)
def _spmm_sc(x_flat, rows_hbm, cols_hbm, vals_hbm, out_hbm,
             acc, G, CI, RI, VB, semE, semG, semS):
    c = lax.axis_index("c")
    s = lax.axis_index("s")

    def eload_at(o, b):
        base = s * EDGES_PER_TILE + b * BLK_E
        return [
            (vals_hbm.at[pl.ds(base, BLK_E)], VB[o]),
            (rows_hbm.at[pl.ds(base, BLK_E)], RI[o]),
            (cols_hbm.at[pl.ds(base, BLK_E)], CI[o]),
        ]

    def eload(o, b):
        for src, dst in eload_at(o, b):
            pltpu.async_copy(src, dst, semE[o])

    def ewait(o, b):
        for src, dst in eload_at(o, b):
            pltpu.make_async_copy(src, dst, semE[o]).wait()

    def transform(o, p):
        ci = CI[o]

        def tfb(i, _):
            sl = pl.ds(i * LANES, LANES)
            ci[sl] = ci[sl] * NPART + p
            return 0
        lax.fori_loop(0, BLK_E // LANES, tfb, 0)

    def gfire(o):
        pltpu.async_copy(x_flat.at[CI[o]], G[o], semG[o])

    def gwait(o):
        pltpu.make_async_copy(x_flat.at[CI[o]], G[o], semG[o]).wait()

    def scale(o):
        g = G[o]
        vb = VB[o]

        @plsc.parallel_loop(0, BLK_E, step=1, unroll=8)
        def _(e):
            bval = plsc.load_gather(vb, [jnp.full((LANES,), e, jnp.int32)])
            g[e, :] = g[e, :] * bval

    def scat_fire(o):
        pltpu.async_copy(G[o], acc.at[RI[o]], semS[o], add=True)

    def scat_wait(o):
        pltpu.make_async_copy(G[o], acc.at[RI[o]], semS[o]).wait()

    for p_local in range(PARTS_PER_CORE):
        p = c * PARTS_PER_CORE + p_local

        def zfill(i, _):
            G[0][i, :] = jnp.zeros((LANES,), jnp.float32)
            return 0
        lax.fori_loop(0, BLK_E, zfill, 0)
        for z in range(ZCOPIES):
            pltpu.sync_copy(
                G[0], acc.at[pl.ds(s * ROWS_PER_TILE + z * BLK_E, BLK_E)])
        pltpu.sync_copy(
            G[0].at[pl.ds(0, ZREM)],
            acc.at[pl.ds(s * ROWS_PER_TILE + ZCOPIES * BLK_E, ZREM)])
        plsc.subcore_barrier()

        def position(b, o, warm):
            o1 = (b + 1) % 4
            o2 = (b + 2) % 4
            if warm:
                scat_wait(o2)
            if b + 2 < NBLOCK:
                eload(o2, b + 2)
            if b + 1 < NBLOCK:
                ewait(o1, b + 1)
                transform(o1, p)
                gfire(o1)
            gwait(o)
            scale(o)
            scat_fire(o)

        eload(0, 0)
        eload(1, 1)
        ewait(0, 0)
        transform(0, p)
        gfire(0)
        for b in range(4):
            position(b, b % 4, b >= 2)

        def steady(m, _):
            for o in range(4):
                b = 4 + 4 * m + o
                scat_wait((o + 2) % 4)
                eload((o + 2) % 4, b + 2)
                ewait((o + 1) % 4, b + 1)
                transform((o + 1) % 4, p)
                gfire((o + 1) % 4)
                gwait(o)
                scale(o)
                scat_fire(o)
            return 0
        lax.fori_loop(0, STEADY, steady, 0)

        for b in range(NBLOCK - TAIL, NBLOCK):
            position(b, b % 4, True)
        scat_wait((NBLOCK - 2) % 4)
        scat_wait((NBLOCK - 1) % 4)
        plsc.subcore_barrier()

        pltpu.sync_copy(acc.at[pl.ds(s * ROWS_PER_TILE, ROWS_PER_TILE)],
                        out_hbm.at[pl.ds(s * ROWS_PER_TILE, ROWS_PER_TILE),
                                   pl.ds(p * LANES, LANES)])
        plsc.subcore_barrier()


def _spmm(x, rows, cols, vals):
    return _spmm_sc(x.reshape(N * NPART, LANES), rows, cols, vals)


BLK = 400
NBLK = N // BLK


def _layer_tc_body(alpha_ref, ego_ref, h1_ref, h2_ref, h3_ref,
                   wgc_ref, bgc_ref, wbi_ref, bbi_ref, act_ref, out_ref):
    a0 = alpha_ref[0, 0]
    a1 = alpha_ref[0, 1]
    a2 = alpha_ref[0, 2]
    m = jnp.maximum(jnp.maximum(a0, a1), a2)
    e0 = jnp.exp(a0 - m)
    e1 = jnp.exp(a1 - m)
    e2 = jnp.exp(a2 - m)
    tot = e0 + e1 + e2
    b0 = e0 / tot
    b1 = e1 / tot
    b2 = e2 / tot
    side = b0 * h1_ref[...] + b1 * h2_ref[...] + b2 * h3_ref[...]
    ego = ego_ref[...]
    sum_e = jnp.dot(side, wgc_ref[...], preferred_element_type=jnp.float32) + bgc_ref[...]
    bi = jnp.dot(ego * side, wbi_ref[...], preferred_element_type=jnp.float32) + bbi_ref[...]
    act = jnp.where(sum_e >= 0, sum_e, 0.2 * sum_e) + bi
    act_ref[...] = act
    nrm = jnp.sqrt(jnp.sum(act * act, axis=1, keepdims=True))
    out_ref[...] = act / jnp.maximum(nrm, 1e-12)


def _layer_tc(alpha, ego, h1, h2, h3, wgc, bgc, wbi, bbi):
    return pl.pallas_call(
        _layer_tc_body,
        grid=(NBLK,),
        in_specs=[
            pl.BlockSpec(memory_space=pltpu.SMEM),
            pl.BlockSpec((BLK, D), lambda i: (i, 0)),
            pl.BlockSpec((BLK, D), lambda i: (i, 0)),
            pl.BlockSpec((BLK, D), lambda i: (i, 0)),
            pl.BlockSpec((BLK, D), lambda i: (i, 0)),
            pl.BlockSpec((D, D), lambda i: (0, 0)),
            pl.BlockSpec((1, D), lambda i: (0, 0)),
            pl.BlockSpec((D, D), lambda i: (0, 0)),
            pl.BlockSpec((1, D), lambda i: (0, 0)),
        ],
        out_specs=[pl.BlockSpec((BLK, D), lambda i: (i, 0)),
                   pl.BlockSpec((BLK, D), lambda i: (i, 0))],
        out_shape=[jax.ShapeDtypeStruct((N, D), jnp.float32),
                   jax.ShapeDtypeStruct((N, D), jnp.float32)],
    )(alpha.reshape(1, 3), ego, h1, h2, h3, wgc, bgc, wbi, bbi)


B3 = 3072
B_PER_W = B3 // (NC * NS)


@functools.partial(
    pl.kernel,
    out_type=[jax.ShapeDtypeStruct((B3, D), jnp.float32)] * 3,
    mesh=_mesh,
    compiler_params=_sc_params,
    scratch_types=[
        pltpu.VMEM((B_PER_W,), jnp.int32),
        pltpu.VMEM((B_PER_W, D), jnp.float32),
        pltpu.VMEM((B_PER_W, D), jnp.float32),
        pltpu.VMEM((B_PER_W, D), jnp.float32),
        pltpu.SemaphoreType.DMA,
    ],
)
def _batch_gather_sc(idx_hbm, t0, t1, t2, o0, o1, o2,
                     idx_v, r0, r1, r2, sem):
    c = lax.axis_index("c")
    s = lax.axis_index("s")
    wid = s * NC + c
    base = wid * B_PER_W
    pltpu.sync_copy(idx_hbm.at[pl.ds(base, B_PER_W)], idx_v)
    cp0 = pltpu.async_copy(t0.at[idx_v], r0, sem)
    cp1 = pltpu.async_copy(t1.at[idx_v], r1, sem)
    cp2 = pltpu.async_copy(t2.at[idx_v], r2, sem)
    cp0.wait()
    cp1.wait()
    cp2.wait()
    pltpu.sync_copy(r0, o0.at[pl.ds(base, B_PER_W)])
    pltpu.sync_copy(r1, o1.at[pl.ds(base, B_PER_W)])
    pltpu.sync_copy(r2, o2.at[pl.ds(base, B_PER_W)])


@jax.jit
def kernel(adj_rows, adj_cols, adj_vals, feature_dense, user_emb, item_emb1,
           item_emb2, alpha0, alpha1, W_gc_0, b_gc_0, W_bi_0, b_bi_0,
           W_gc_1, b_gc_1, W_bi_1, b_bi_1, users, pos_items, neg_items):
    ego0 = jnp.concatenate(
        [jnp.concatenate([user_emb, item_emb1], axis=0),
         jnp.concatenate([feature_dense, item_emb2], axis=0)], axis=1)

    alphas = [alpha0, alpha1]
    weights = [(W_gc_0, b_gc_0, W_bi_0, b_bi_0),
               (W_gc_1, b_gc_1, W_bi_1, b_bi_1)]
    ego = ego0
    embs = [ego0]
    for k in range(2):
        h1 = _spmm(ego, adj_rows, adj_cols, adj_vals)
        h2 = _spmm(h1, adj_rows, adj_cols, adj_vals)
        h3 = _spmm(h2, adj_rows, adj_cols, adj_vals)
        wgc, bgc, wbi, bbi = weights[k]
        ego, norm = _layer_tc(alphas[k], ego, h1, h2, h3, wgc, bgc, wbi, bbi)
        embs.append(norm)

    idx3 = jnp.concatenate(
        [users, N_USER + pos_items, N_USER + neg_items], axis=0)
    o0, o1, o2 = _batch_gather_sc(idx3, embs[0], embs[1], embs[2])
    return jnp.concatenate([o0, o1, o2], axis=1)
